# Initial kernel scaffold; baseline (speedup 1.0000x reference)
import functools

import jax
import jax.numpy as jnp
from jax import lax
from jax.experimental import pallas as pl
from jax.experimental.pallas import tpu as pltpu
from jax.experimental.pallas import tpu_sc as plsc

N = 4096
HID = 256
E = 262144

NC = 2
NS = 16
LANES = 16

EPT = E // NS
CH = 2048
NBLK = 8
RB = N // (NC * NBLK)
BLK_W = RB * N
TPW = BLK_W // NS


def _tc_scores(emb, w_pad, b_pad):
    def body(emb_ref, w_ref, b_ref, out_ref):
        out_ref[...] = (
            jnp.dot(emb_ref[...], w_ref[...], preferred_element_type=jnp.float32)
            + b_ref[...]
        )

    return pl.pallas_call(
        body,
        out_shape=jax.ShapeDtypeStruct((N, 128), jnp.float32),
    )(emb, w_pad, b_pad)


def _sc_scatter(s1, s2, v0, v1):
    mesh = plsc.VectorSubcoreMesh(core_axis_name="c", subcore_axis_name="s")

    @functools.partial(
        pl.kernel,
        out_type=jax.ShapeDtypeStruct((N * N,), jnp.float32),
        mesh=mesh,
        scratch_types=[
            pltpu.VMEM((N,), jnp.float32),
            pltpu.VMEM((N,), jnp.float32),
            pltpu.VMEM((EPT,), jnp.int32),
            pltpu.VMEM((EPT,), jnp.int32),
            pltpu.VMEM((EPT,), jnp.int32),
            pltpu.VMEM((EPT,), jnp.float32),
            pltpu.VMEM((CH,), jnp.int32),
            pltpu.VMEM((CH,), jnp.float32),
            pltpu.VMEM((TPW // 4,), jnp.float32),
            pltpu.VMEM_SHARED((BLK_W,), jnp.float32),
        ],
    )
    def k(s1_hbm, s2_hbm, v0_hbm, v1_hbm, z_hbm,
          s1_v, s2_v, vi0, vi1, g_v, val_v, st_i, st_v, zb, acc):
        cid = lax.axis_index("c")
        sid = lax.axis_index("s")

        pltpu.sync_copy(s1_hbm, s1_v)
        pltpu.sync_copy(s2_hbm, s2_v)
        pltpu.sync_copy(v0_hbm.at[pl.ds(sid * EPT, EPT)], vi0)
        pltpu.sync_copy(v1_hbm.at[pl.ds(sid * EPT, EPT)], vi1)

        zvec = jnp.zeros((LANES,), jnp.float32)

        def zero_body(i, _):
            zb[pl.ds(i * LANES, LANES)] = zvec
            return 0

        lax.fori_loop(0, (TPW // 4) // LANES, zero_body, 0)

        def pre_body(i, _):
            a = vi0[pl.ds(i * LANES, LANES)]
            c = vi1[pl.ds(i * LANES, LANES)]
            x1 = plsc.load_gather(s1_v, [a])
            x2 = plsc.load_gather(s2_v, [c])
            g_v[pl.ds(i * LANES, LANES)] = (a << 12) + c
            val_v[pl.ds(i * LANES, LANES)] = x1 + x2
            return 0

        lax.fori_loop(0, EPT // LANES, pre_body, 0)

        for blk in range(NBLK):
            for j in range(4):
                pltpu.sync_copy(
                    zb, acc.at[pl.ds(sid * TPW + j * (TPW // 4), TPW // 4)]
                )
            plsc.subcore_barrier()

            base = (cid * NBLK + blk) * BLK_W

            def chunk_body(ci, _):
                def vec_body(i, _):
                    off = ci * CH + i * LANES
                    idx16 = g_v[pl.ds(off, LANES)] - base
                    v16 = val_v[pl.ds(off, LANES)]
                    inr = plsc.bitcast(idx16, jnp.uint32) < jnp.uint32(BLK_W)
                    st_i[pl.ds(i * LANES, LANES)] = idx16 & (BLK_W - 1)
                    st_v[pl.ds(i * LANES, LANES)] = jnp.where(inr, v16, 0.0)
                    return 0

                lax.fori_loop(0, CH // LANES, vec_body, 0)
                pltpu.sync_copy(st_v, acc.at[st_i], add=True)
                return 0

            lax.fori_loop(0, EPT // CH, chunk_body, 0)
            plsc.subcore_barrier()

            zoff = base + sid * TPW
            pltpu.sync_copy(acc.at[pl.ds(sid * TPW, TPW)], z_hbm.at[pl.ds(zoff, TPW)])
            plsc.subcore_barrier()

    return k(s1, s2, v0, v1)


def kernel(embeddings, v_indices, W, b):
    w_pad = jnp.zeros((2 * HID, 128), jnp.float32)
    w_pad = w_pad.at[:HID, 0].set(W[0, :HID])
    w_pad = w_pad.at[HID:, 1].set(W[0, HID:])
    b_pad = jnp.zeros((1, 128), jnp.float32).at[0, 0].set(b[0])

    scores = _tc_scores(embeddings, w_pad, b_pad)
    s1 = scores[:, 0]
    s2 = scores[:, 1]

    zflat = _sc_scatter(s1, s2, v_indices[0], v_indices[1])
    return zflat.reshape(N, N)


# speedup vs baseline: 7.4370x; 7.4370x over previous
import functools

import jax
import jax.numpy as jnp
from jax import lax
from jax.experimental import pallas as pl
from jax.experimental.pallas import tpu as pltpu
from jax.experimental.pallas import tpu_sc as plsc

N = 4096
HID = 256
E = 262144

NC = 2
NS = 16
LANES = 16

EPT = E // NS
CH = 2048
NBLK = 8
RB = N // (NC * NBLK)
BLK_W = RB * N
TPW = BLK_W // NS


def _tc_scores(emb, w_pad, b_pad):
    def body(emb_ref, w_ref, b_ref, out_ref):
        out_ref[...] = (
            jnp.dot(emb_ref[...], w_ref[...], preferred_element_type=jnp.float32)
            + b_ref[...]
        )

    return pl.pallas_call(
        body,
        out_shape=jax.ShapeDtypeStruct((N, 128), jnp.float32),
    )(emb, w_pad, b_pad)


def _sc_scatter(s1, s2, v0, v1):
    mesh = plsc.VectorSubcoreMesh(core_axis_name="c", subcore_axis_name="s")

    @functools.partial(
        pl.kernel,
        out_type=jax.ShapeDtypeStruct((N * N,), jnp.float32),
        mesh=mesh,
        scratch_types=[
            pltpu.VMEM((CH,), jnp.int32),
            pltpu.VMEM((CH,), jnp.int32),
            pltpu.VMEM((CH,), jnp.float32),
            pltpu.VMEM((CH,), jnp.float32),
            pltpu.VMEM((EPT,), jnp.int32),
            pltpu.VMEM((EPT,), jnp.float32),
            pltpu.VMEM((CH,), jnp.int32),
            pltpu.VMEM((CH,), jnp.float32),
            pltpu.VMEM((4096,), jnp.float32),
            pltpu.VMEM_SHARED((BLK_W,), jnp.float32),
        ],
    )
    def k(s1_hbm, s2_hbm, v0_hbm, v1_hbm, z_hbm,
          vi0c, vi1c, x1c, x2c, g_v, val_v, st_i, st_v, zb, acc):
        cid = lax.axis_index("c")
        sid = lax.axis_index("s")

        zvec = jnp.zeros((LANES,), jnp.float32)

        def zero_body(i, _):
            zb[pl.ds(i * LANES, LANES)] = zvec
            return 0

        lax.fori_loop(0, 4096 // LANES, zero_body, 0)

        def pre_chunk(ci, _):
            ebase = sid * EPT + ci * CH
            pltpu.sync_copy(v0_hbm.at[pl.ds(ebase, CH)], vi0c)
            pltpu.sync_copy(v1_hbm.at[pl.ds(ebase, CH)], vi1c)
            pltpu.sync_copy(s1_hbm.at[vi0c], x1c)
            pltpu.sync_copy(s2_hbm.at[vi1c], x2c)

            def pre_body(i, _):
                a = vi0c[pl.ds(i * LANES, LANES)]
                c = vi1c[pl.ds(i * LANES, LANES)]
                g_v[pl.ds(ci * CH + i * LANES, LANES)] = (a << 12) + c
                val_v[pl.ds(ci * CH + i * LANES, LANES)] = (
                    x1c[pl.ds(i * LANES, LANES)] + x2c[pl.ds(i * LANES, LANES)]
                )
                return 0

            lax.fori_loop(0, CH // LANES, pre_body, 0)
            return 0

        lax.fori_loop(0, EPT // CH, pre_chunk, 0)

        for blk in range(NBLK):
            for j in range(TPW // 4096):
                pltpu.sync_copy(
                    zb, acc.at[pl.ds(sid * TPW + j * 4096, 4096)]
                )
            plsc.subcore_barrier()

            base = (cid * NBLK + blk) * BLK_W

            def chunk_body(ci, _):
                def vec_body(i, _):
                    off = ci * CH + i * LANES
                    idx16 = g_v[pl.ds(off, LANES)] - base
                    v16 = val_v[pl.ds(off, LANES)]
                    inr = plsc.bitcast(idx16, jnp.uint32) < jnp.uint32(BLK_W)
                    st_i[pl.ds(i * LANES, LANES)] = idx16 & (BLK_W - 1)
                    st_v[pl.ds(i * LANES, LANES)] = jnp.where(inr, v16, 0.0)
                    return 0

                lax.fori_loop(0, CH // LANES, vec_body, 0)
                pltpu.sync_copy(st_v, acc.at[st_i], add=True)
                return 0

            lax.fori_loop(0, EPT // CH, chunk_body, 0)
            plsc.subcore_barrier()

            zoff = base + sid * TPW
            pltpu.sync_copy(acc.at[pl.ds(sid * TPW, TPW)], z_hbm.at[pl.ds(zoff, TPW)])
            plsc.subcore_barrier()

    return k(s1, s2, v0, v1)


def kernel(embeddings, v_indices, W, b):
    w_pad = jnp.zeros((HID, 128), jnp.float32)
    w_pad = w_pad.at[:, 0].set(W[0, :HID])
    w_pad = w_pad.at[:, 1].set(W[0, HID:])
    b_pad = jnp.zeros((1, 128), jnp.float32).at[0, 0].set(b[0])

    scores = _tc_scores(embeddings, w_pad, b_pad)
    s1 = scores[:, 0]
    s2 = scores[:, 1]

    zflat = _sc_scatter(s1, s2, v_indices[0], v_indices[1])
    return zflat.reshape(N, N)


# speedup vs baseline: 10.1409x; 1.3636x over previous
import functools

import jax
import jax.numpy as jnp
from jax import lax
from jax.experimental import pallas as pl
from jax.experimental.pallas import tpu as pltpu
from jax.experimental.pallas import tpu_sc as plsc

N = 4096
HID = 256
E = 262144

NC = 2
NS = 16
LANES = 16

EPT = E // NS
CH = 2048
NCH = EPT // CH
NBLK = 8
RB = N // (NC * NBLK)
BLK_W = RB * N
TPW = BLK_W // NS
ZB = 4096


def _tc_scores(emb, w_pad, b_pad):
    def body(emb_ref, w_ref, b_ref, out_ref):
        out_ref[...] = (
            jnp.dot(emb_ref[...], w_ref[...], preferred_element_type=jnp.float32)
            + b_ref[...]
        )

    return pl.pallas_call(
        body,
        out_shape=jax.ShapeDtypeStruct((N, 128), jnp.float32),
    )(emb, w_pad, b_pad)


def _sc_scatter(s1, s2, v0, v1):
    mesh = plsc.VectorSubcoreMesh(core_axis_name="c", subcore_axis_name="s")

    @functools.partial(
        pl.kernel,
        out_type=jax.ShapeDtypeStruct((N * N,), jnp.float32),
        mesh=mesh,
        scratch_types=[
            pltpu.VMEM((CH,), jnp.int32),
            pltpu.VMEM((CH,), jnp.int32),
            pltpu.VMEM((CH,), jnp.int32),
            pltpu.VMEM((CH,), jnp.int32),
            pltpu.VMEM((CH,), jnp.float32),
            pltpu.VMEM((CH,), jnp.float32),
            pltpu.VMEM((CH,), jnp.float32),
            pltpu.VMEM((CH,), jnp.float32),
            pltpu.VMEM((EPT,), jnp.int32),
            pltpu.VMEM((EPT,), jnp.float32),
            pltpu.VMEM((CH,), jnp.int32),
            pltpu.VMEM((CH,), jnp.int32),
            pltpu.VMEM((CH,), jnp.float32),
            pltpu.VMEM((CH,), jnp.float32),
            pltpu.VMEM((ZB,), jnp.float32),
            pltpu.VMEM_SHARED((BLK_W,), jnp.float32),
            pltpu.SemaphoreType.DMA,
            pltpu.SemaphoreType.DMA,
            pltpu.SemaphoreType.DMA,
            pltpu.SemaphoreType.DMA,
            pltpu.SemaphoreType.DMA,
            pltpu.SemaphoreType.DMA,
            pltpu.SemaphoreType.DMA,
        ],
    )
    def k(s1_hbm, s2_hbm, v0_hbm, v1_hbm, z_hbm,
          vi0a, vi0b, vi1a, vi1b, x1a, x1b, x2a, x2b, g_v, val_v,
          sia, sib, sva, svb, zb, acc,
          sem_v0, sem_v1, sem_x1, sem_x2, sem_s0, sem_s1, sem_z):
        cid = lax.axis_index("c")
        sid = lax.axis_index("s")
        sem_s = [sem_s0, sem_s1]
        vi0c = [vi0a, vi0b]
        vi1c = [vi1a, vi1b]
        x1c = [x1a, x1b]
        x2c = [x2a, x2b]
        st_i = [sia, sib]
        st_v = [sva, svb]

        zvec = jnp.zeros((LANES,), jnp.float32)

        def zero_body(i, _):
            zb[pl.ds(i * LANES, LANES)] = zvec
            return 0

        lax.fori_loop(0, ZB // LANES, zero_body, 0)

        def vstart(ci):
            s = ci % 2
            ebase = sid * EPT + ci * CH
            h0 = pltpu.async_copy(v0_hbm.at[pl.ds(ebase, CH)], vi0c[s], sem_v0)
            h1 = pltpu.async_copy(v1_hbm.at[pl.ds(ebase, CH)], vi1c[s], sem_v1)
            return (h0, h1)

        def gstart(ci):
            s = ci % 2
            h0 = pltpu.async_copy(s1_hbm.at[vi0c[s]], x1c[s], sem_x1)
            h1 = pltpu.async_copy(s2_hbm.at[vi1c[s]], x2c[s], sem_x2)
            return (h0, h1)

        def compute(ci):
            s = ci % 2

            def pre_body(i, _):
                a = vi0c[s][pl.ds(i * LANES, LANES)]
                c = vi1c[s][pl.ds(i * LANES, LANES)]
                g_v[pl.ds(ci * CH + i * LANES, LANES)] = (a << 12) + c
                val_v[pl.ds(ci * CH + i * LANES, LANES)] = (
                    x1c[s][pl.ds(i * LANES, LANES)]
                    + x2c[s][pl.ds(i * LANES, LANES)]
                )
                return 0

            lax.fori_loop(0, CH // LANES, pre_body, 0)

        vh = {0: vstart(0)}
        gh = {}
        for ci in range(NCH):
            if ci == 0:
                for h in vh.pop(0):
                    h.wait()
                gh[0] = gstart(0)
                vh[1] = vstart(1)
            for h in gh.pop(ci):
                h.wait()
            if ci + 1 < NCH:
                for h in vh.pop(ci + 1):
                    h.wait()
                gh[ci + 1] = gstart(ci + 1)
            compute(ci)
            if ci + 2 < NCH:
                vh[ci + 2] = vstart(ci + 2)

        zh = []
        for j in range(TPW // ZB):
            zh.append(
                pltpu.async_copy(zb, acc.at[pl.ds(sid * TPW + j * ZB, ZB)], sem_z)
            )
        for h in zh:
            h.wait()
        plsc.subcore_barrier()

        for blk in range(NBLK):
            base = (cid * NBLK + blk) * BLK_W

            sh = [None, None]
            for ci in range(NCH):
                s = ci % 2

                def vec_body(i, _):
                    off = ci * CH + i * LANES
                    idx16 = g_v[pl.ds(off, LANES)] - base
                    v16 = val_v[pl.ds(off, LANES)]
                    inr = plsc.bitcast(idx16, jnp.uint32) < jnp.uint32(BLK_W)
                    st_i[s][pl.ds(i * LANES, LANES)] = idx16 & (BLK_W - 1)
                    st_v[s][pl.ds(i * LANES, LANES)] = jnp.where(inr, v16, 0.0)
                    return 0

                if sh[s] is not None:
                    sh[s].wait()
                lax.fori_loop(0, CH // LANES, vec_body, 0)
                sh[s] = pltpu.async_copy(
                    st_v[s], acc.at[st_i[s]], sem_s[s], add=True
                )
            for h in sh:
                h.wait()
            plsc.subcore_barrier()

            pltpu.sync_copy(
                acc.at[pl.ds(sid * TPW, TPW)],
                z_hbm.at[pl.ds(base + sid * TPW, TPW)],
            )
            if blk + 1 < NBLK:
                zh = []
                for j in range(TPW // ZB):
                    zh.append(
                        pltpu.async_copy(
                            zb, acc.at[pl.ds(sid * TPW + j * ZB, ZB)], sem_z
                        )
                    )
                for h in zh:
                    h.wait()
                plsc.subcore_barrier()

    return k(s1, s2, v0, v1)


def kernel(embeddings, v_indices, W, b):
    w_pad = jnp.zeros((HID, 128), jnp.float32)
    w_pad = w_pad.at[:, 0].set(W[0, :HID])
    w_pad = w_pad.at[:, 1].set(W[0, HID:])
    b_pad = jnp.zeros((1, 128), jnp.float32).at[0, 0].set(b[0])

    scores = _tc_scores(embeddings, w_pad, b_pad)
    s1 = scores[:, 0]
    s2 = scores[:, 1]

    zflat = _sc_scatter(s1, s2, v_indices[0], v_indices[1])
    return zflat.reshape(N, N)
